# baseline (speedup 1.0000x reference)
import functools

import jax
import jax.numpy as jnp
from jax import lax
from jax.experimental import pallas as pl
from jax.experimental.pallas import tpu as pltpu
from jax.experimental.pallas import tpu_sc as plsc

_D = 64
_B = 16384
_L = 16
_NC = 2
_NS = 16
_NW = _NC * _NS
_BPW = _B // _NW


def _sc_body(idx_hbm, inp_hbm, w_hbm, out_hbm, idx_v, rows_v, inp_v, sem):
    wid = lax.axis_index("s") * _NC + lax.axis_index("c")
    base = wid * _BPW
    pltpu.sync_copy(idx_hbm.at[pl.ds(base, _BPW)], idx_v)
    gather = pltpu.async_copy(w_hbm.at[idx_v], rows_v, sem)
    pltpu.sync_copy(inp_hbm.at[pl.ds(base, _BPW)], inp_v)
    gather.wait()

    def row(i, carry):
        for c in range(_D // _L):
            s = pl.ds(c * _L, _L)
            rows_v[i, s] = rows_v[i, s] * inp_v[i, s]
        return carry

    lax.fori_loop(0, _BPW, row, 0)
    pltpu.sync_copy(rows_v, out_hbm.at[pl.ds(base, _BPW)])


def kernel(idx, input, weight):
    mesh = plsc.VectorSubcoreMesh(core_axis_name="c", subcore_axis_name="s")
    k = functools.partial(
        pl.kernel,
        mesh=mesh,
        out_type=jax.ShapeDtypeStruct((_B, _D), jnp.float32),
        scratch_types=[
            pltpu.VMEM((_BPW,), jnp.int32),
            pltpu.VMEM((_BPW, _D), jnp.float32),
            pltpu.VMEM((_BPW, _D), jnp.float32),
            pltpu.SemaphoreType.DMA,
        ],
        compiler_params=pltpu.CompilerParams(use_tc_tiling_on_sc=False),
    )(_sc_body)
    return k(idx.astype(jnp.int32), input, weight)


# speedup vs baseline: 1.6111x; 1.6111x over previous
import functools

import jax
import jax.numpy as jnp
from jax import lax
from jax.experimental import pallas as pl
from jax.experimental.pallas import tpu as pltpu
from jax.experimental.pallas import tpu_sc as plsc

_D = 64
_B = 16384
_L = 16
_NC = 2
_NS = 16
_NW = _NC * _NS
_BPW = _B // _NW


def _sc_body(idx_hbm, inp_hbm, w_hbm, out_hbm, idx_v, rows_v, inp_v, gsem):
    wid = lax.axis_index("s") * _NC + lax.axis_index("c")
    base = wid * _BPW
    pltpu.sync_copy(idx_hbm.at[pl.ds(base, _BPW)], idx_v)

    def fire(i, carry):
        v = idx_v[pl.ds(i * _L, _L)]
        for j in range(_L):
            pltpu.async_copy(w_hbm.at[v[j]], rows_v.at[i * _L + j], gsem)
        return carry

    lax.fori_loop(0, _BPW // _L, fire, 0)
    pltpu.sync_copy(inp_hbm.at[pl.ds(base * _D, _BPW * _D)], inp_v)
    pltpu.make_async_copy(w_hbm.at[pl.ds(0, _BPW)], rows_v, gsem).wait()

    def row(i, carry):
        for c in range(_D // _L):
            s = pl.ds(i * _D + c * _L, _L)
            inp_v[s] = inp_v[s] * rows_v[i, pl.ds(c * _L, _L)]
        return carry

    lax.fori_loop(0, _BPW, row, 0)
    pltpu.sync_copy(inp_v, out_hbm.at[pl.ds(base * _D, _BPW * _D)])


def kernel(idx, input, weight):
    mesh = plsc.VectorSubcoreMesh(core_axis_name="c", subcore_axis_name="s")
    k = functools.partial(
        pl.kernel,
        mesh=mesh,
        out_type=jax.ShapeDtypeStruct((_B * _D,), jnp.float32),
        scratch_types=[
            pltpu.VMEM((_BPW,), jnp.int32),
            pltpu.VMEM((_BPW, _D), jnp.float32),
            pltpu.VMEM((_BPW * _D,), jnp.float32),
            pltpu.SemaphoreType.DMA,
        ],
    )(_sc_body)
    out = k(idx.astype(jnp.int32), input.reshape(-1), weight)
    return out.reshape(_B, _D)
